# baseline (speedup 1.0000x reference)
import functools

import jax
import jax.numpy as jnp
from jax import lax
from jax.experimental import pallas as pl
from jax.experimental.pallas import tpu as pltpu
from jax.experimental.pallas import tpu_sc as plsc

NUM_CORES = 2
NUM_SUBCORES = 16
LANES = 16
NUM_WORKERS = NUM_CORES * NUM_SUBCORES

BATCH = 16384
EMB_DIM = 64
B_PER_W = BATCH // NUM_WORKERS
ROWS_PER_BLOCK = LANES
NUM_BLOCKS = B_PER_W // ROWS_PER_BLOCK


def _mf_kernel(user_ids_hbm, item_ids_hbm, user_table_hbm, item_table_hbm,
               out_hbm, uidx_v, iidx_v, urows_v, irows_v, out_v,
               sem_u, sem_i):
    wid = lax.axis_index("s") * NUM_CORES + lax.axis_index("c")
    base = wid * B_PER_W

    pltpu.sync_copy(user_ids_hbm.at[pl.ds(base, B_PER_W)], uidx_v)
    pltpu.sync_copy(item_ids_hbm.at[pl.ds(base, B_PER_W)], iidx_v)

    cp_u = pltpu.async_copy(user_table_hbm.at[uidx_v], urows_v, sem_u)
    cp_i = pltpu.async_copy(item_table_hbm.at[iidx_v], irows_v, sem_i)
    cp_u.wait()
    cp_i.wait()

    lane = lax.iota(jnp.int32, LANES)

    def block_body(blk, _):
        rows = blk * ROWS_PER_BLOCK + lane
        acc = jnp.zeros((LANES,), jnp.float32)
        for d in range(EMB_DIM):
            col = jnp.full((LANES,), d, jnp.int32)
            u = plsc.load_gather(urows_v, [rows, col])
            v = plsc.load_gather(irows_v, [rows, col])
            acc = acc + u * v
        out_v[pl.ds(blk * ROWS_PER_BLOCK, ROWS_PER_BLOCK)] = acc
        return ()

    lax.fori_loop(0, NUM_BLOCKS, block_body, (), unroll=False)

    pltpu.sync_copy(out_v, out_hbm.at[pl.ds(base, B_PER_W)])


@jax.jit
def _mf(user_ids, item_ids, user_table, item_table):
    run = pl.kernel(
        _mf_kernel,
        out_type=jax.ShapeDtypeStruct((BATCH,), jnp.float32),
        mesh=plsc.VectorSubcoreMesh(
            core_axis_name="c", subcore_axis_name="s", num_cores=NUM_CORES
        ),
        scratch_types=[
            pltpu.VMEM((B_PER_W,), jnp.int32),
            pltpu.VMEM((B_PER_W,), jnp.int32),
            pltpu.VMEM((B_PER_W, EMB_DIM), jnp.float32),
            pltpu.VMEM((B_PER_W, EMB_DIM), jnp.float32),
            pltpu.VMEM((B_PER_W,), jnp.float32),
            pltpu.SemaphoreType.DMA,
            pltpu.SemaphoreType.DMA,
        ],
        compiler_params=pltpu.CompilerParams(
            needs_layout_passes=False, use_tc_tiling_on_sc=False
        ),
    )
    return run(user_ids, item_ids, user_table, item_table)


def kernel(user_ids, item_ids, user_table, item_table):
    return _mf(user_ids.astype(jnp.int32), item_ids.astype(jnp.int32),
               user_table, item_table)
